# baseline (speedup 1.0000x reference)
import functools

import jax
import jax.numpy as jnp
from jax.experimental import pallas as pl

N = 50000
T = 16
D_IN = 2
EMB = 32
ENC = 64
HEADS = 3
DEC = 128
OUT_LEN = 25
HC = HEADS * ENC
G = 2048


_BE = 512


def _enc_body(x_ref, wip_ref, bip_ref,
              wir_ref, wiz_ref, win_ref,
              whr_ref, whz_ref, whn_ref,
              brz_ref, bin_ref, bhn_ref,
              wdyn_ref, bdyn_ref, out_ref):
    xb = x_ref[...]
    B = xb.shape[0]
    wip = wip_ref[...]
    bip = bip_ref[...]
    h = jnp.zeros((B, ENC), dtype=jnp.float32)
    for t in range(T):
        x0 = xb[:, 2 * t:2 * t + 1]
        x1 = xb[:, 2 * t + 1:2 * t + 2]
        emb = x0 * wip[0:1, :] + x1 * wip[1:2, :] + bip
        emb = jnp.maximum(emb, 0.1 * emb)
        r = jax.nn.sigmoid(jnp.dot(emb, wir_ref[...], preferred_element_type=jnp.float32)
                           + jnp.dot(h, whr_ref[...], preferred_element_type=jnp.float32)
                           + brz_ref[0:1, :])
        z = jax.nn.sigmoid(jnp.dot(emb, wiz_ref[...], preferred_element_type=jnp.float32)
                           + jnp.dot(h, whz_ref[...], preferred_element_type=jnp.float32)
                           + brz_ref[1:2, :])
        hn = jnp.dot(h, whn_ref[...], preferred_element_type=jnp.float32) + bhn_ref[...]
        xn = jnp.dot(emb, win_ref[...], preferred_element_type=jnp.float32) + bin_ref[...]
        n = jnp.tanh(xn + r * hn)
        h = (1.0 - z) * n + z * h
    out = jnp.dot(h, wdyn_ref[...], preferred_element_type=jnp.float32) + bdyn_ref[...]
    out_ref[...] = jnp.maximum(out, 0.1 * out)


def _encoder(x2d, p):
    grid = pl.cdiv(N, _BE)
    wi = p["gru_Wi"]
    wh = p["gru_Wh"]
    bi = p["gru_bi"]
    bh = p["gru_bh"]
    brz = jnp.stack([bi[0:ENC] + bh[0:ENC], bi[ENC:2 * ENC] + bh[ENC:2 * ENC]])
    full = lambda s: pl.BlockSpec(s, lambda i: tuple(0 for _ in s))
    return pl.pallas_call(
        _enc_body,
        grid=(grid,),
        in_specs=[
            pl.BlockSpec((_BE, T * D_IN), lambda i: (i, 0)),
            full((D_IN, EMB)), full((1, EMB)),
            full((EMB, ENC)), full((EMB, ENC)), full((EMB, ENC)),
            full((ENC, ENC)), full((ENC, ENC)), full((ENC, ENC)),
            full((2, ENC)), full((1, ENC)), full((1, ENC)),
            full((ENC, ENC)), full((1, ENC)),
        ],
        out_specs=pl.BlockSpec((_BE, ENC), lambda i: (i, 0)),
        out_shape=jax.ShapeDtypeStruct((N, ENC), jnp.float32),
    )(x2d, p["W_ip"], p["b_ip"][None, :],
      wi[:, 0:ENC], wi[:, ENC:2 * ENC], wi[:, 2 * ENC:],
      wh[:, 0:ENC], wh[:, ENC:2 * ENC], wh[:, 2 * ENC:],
      brz, bi[None, 2 * ENC:], bh[None, 2 * ENC:],
      p["W_dyn"], p["b_dyn"][None, :])


_BG = 1024


def _gat_dense_body(x_ref, w_ref, a_ref, xl_ref, sc_ref):
    xl = jnp.dot(x_ref[...], w_ref[...], preferred_element_type=jnp.float32)
    xl_ref[...] = xl
    sc_ref[...] = jnp.dot(xl, a_ref[...], preferred_element_type=jnp.float32)


def _gat_dense(x, w, a_src, a_dst):
    d_in = x.shape[1]
    amat = jnp.zeros((HC, 8), jnp.float32)
    for h in range(HEADS):
        amat = amat.at[h * ENC:(h + 1) * ENC, h].set(a_src[h])
        amat = amat.at[h * ENC:(h + 1) * ENC, h + 4].set(a_dst[h])
    grid = pl.cdiv(N, _BG)
    return pl.pallas_call(
        _gat_dense_body,
        grid=(grid,),
        in_specs=[
            pl.BlockSpec((_BG, d_in), lambda i: (i, 0)),
            pl.BlockSpec((d_in, HC), lambda i: (0, 0)),
            pl.BlockSpec((HC, 8), lambda i: (0, 0)),
        ],
        out_specs=[
            pl.BlockSpec((_BG, HC), lambda i: (i, 0)),
            pl.BlockSpec((_BG, 8), lambda i: (i, 0)),
        ],
        out_shape=[
            jax.ShapeDtypeStruct((N, HC), jnp.float32),
            jax.ShapeDtypeStruct((N, 8), jnp.float32),
        ],
    )(x, w, amat)


def _gat_layer(x, src, dst, w, a_src, a_dst, bias, dst_seg=None, nseg=N):
    xl, sc = _gat_dense(x, w, a_src, a_dst)
    s_src = sc[:, 0:HEADS]
    s_dst = sc[:, 4:4 + HEADS]
    seg = dst if dst_seg is None else dst_seg
    alpha = s_src[src] + s_dst[dst]
    alpha = jnp.maximum(alpha, 0.2 * alpha)
    w_e = jnp.exp(alpha)
    den = jax.ops.segment_sum(w_e, seg, num_segments=nseg)
    msg = w_e[:, :, None] * xl[src].reshape(-1, HEADS, ENC)
    num = jax.ops.segment_sum(msg.reshape(-1, HC), seg, num_segments=nseg)
    out = num.reshape(nseg, HEADS, ENC) / (den[:, :, None] + 1e-16)
    return out.reshape(nseg, HC) + bias


_BD = 512


def _dec_body(xt_ref, wfc_ref, bfc_ref,
              wi0_ref, wh0_ref, b0_ref,
              wi1_ref, wh1_ref, b1_ref,
              wop_ref, bop_ref, out_ref):
    B = xt_ref.shape[0]
    enc = jnp.dot(xt_ref[...], wfc_ref[...], preferred_element_type=jnp.float32) + bfc_ref[...]
    enc = jnp.maximum(enc, 0.1 * enc)
    gx0 = jnp.dot(enc, wi0_ref[...], preferred_element_type=jnp.float32) + b0_ref[...]
    h1 = jnp.zeros((B, DEC), jnp.float32)
    c1 = jnp.zeros((B, DEC), jnp.float32)
    h2 = jnp.zeros((B, DEC), jnp.float32)
    c2 = jnp.zeros((B, DEC), jnp.float32)
    wop = wop_ref[...]
    bop = bop_ref[...]
    for t in range(OUT_LEN):
        g = gx0 + jnp.dot(h1, wh0_ref[...], preferred_element_type=jnp.float32)
        i = jax.nn.sigmoid(g[:, 0:DEC])
        f = jax.nn.sigmoid(g[:, DEC:2 * DEC])
        gg = jnp.tanh(g[:, 2 * DEC:3 * DEC])
        o = jax.nn.sigmoid(g[:, 3 * DEC:])
        c1 = f * c1 + i * gg
        h1 = o * jnp.tanh(c1)
        g = (jnp.dot(h1, wi1_ref[...], preferred_element_type=jnp.float32)
             + jnp.dot(h2, wh1_ref[...], preferred_element_type=jnp.float32) + b1_ref[...])
        i = jax.nn.sigmoid(g[:, 0:DEC])
        f = jax.nn.sigmoid(g[:, DEC:2 * DEC])
        gg = jnp.tanh(g[:, 2 * DEC:3 * DEC])
        o = jax.nn.sigmoid(g[:, 3 * DEC:])
        c2 = f * c2 + i * gg
        h2 = o * jnp.tanh(c2)
        out_ref[:, pl.ds(2 * t, 2)] = (
            jnp.dot(h2, wop, preferred_element_type=jnp.float32) + bop)


def _decoder(g2t, p):
    grid = pl.cdiv(G, _BD)
    full = lambda s: pl.BlockSpec(s, lambda i: tuple(0 for _ in s))
    out = pl.pallas_call(
        _dec_body,
        grid=(grid,),
        in_specs=[
            pl.BlockSpec((_BD, HC), lambda i: (i, 0)),
            full((HC, ENC)), full((1, ENC)),
            full((ENC, 4 * DEC)), full((DEC, 4 * DEC)), full((1, 4 * DEC)),
            full((DEC, 4 * DEC)), full((DEC, 4 * DEC)), full((1, 4 * DEC)),
            full((DEC, 2)), full((1, 2)),
        ],
        out_specs=pl.BlockSpec((_BD, 2 * OUT_LEN), lambda i: (i, 0)),
        out_shape=jax.ShapeDtypeStruct((G, 2 * OUT_LEN), jnp.float32),
    )(g2t, p["W_fc"], p["b_fc"][None, :],
      p["lstm0_Wi"], p["lstm0_Wh"], p["lstm0_b"][None, :],
      p["lstm1_Wi"], p["lstm1_Wh"], p["lstm1_b"][None, :],
      p["W_op"], p["b_op"][None, :])
    return out.reshape(G, OUT_LEN, 2)


def kernel(x, params, edge_index, batch, num_graphs):
    p = params
    x2d = x.reshape(N, T * D_IN)
    henc = _encoder(x2d, p)

    ei = edge_index.astype(jnp.int32)
    loop = jnp.arange(N, dtype=jnp.int32)
    src = jnp.concatenate([ei[0], loop])
    dst = jnp.concatenate([ei[1], loop])

    g1 = _gat_layer(henc, src, dst, p["gat1_W"], p["gat1_asrc"], p["gat1_adst"], p["gat1_b"])

    b32 = batch.astype(jnp.int32)
    is_tgt = jnp.concatenate([jnp.ones((1,), jnp.bool_), b32[1:] != b32[:-1]])
    slot = jnp.where(is_tgt, b32, G)
    dst_seg = slot[dst]
    g2t = _gat_layer(g1, src, dst, p["gat2_W"], p["gat2_asrc"], p["gat2_adst"],
                     p["gat2_b"], dst_seg=dst_seg, nseg=G + 1)[:G]
    return _decoder(g2t, p)


# speedup vs baseline: 4.5065x; 4.5065x over previous
import functools

import jax
import jax.numpy as jnp
from jax import lax
from jax.experimental import pallas as pl
from jax.experimental.pallas import tpu as pltpu
from jax.experimental.pallas import tpu_sc as plsc

N = 50000
T = 16
D_IN = 2
EMB = 32
ENC = 64
HEADS = 3
DEC = 128
OUT_LEN = 25
HC = HEADS * ENC
G = 2048


_BE = 512


def _enc_body(x_ref, wip_ref, bip_ref,
              wir_ref, wiz_ref, win_ref,
              whr_ref, whz_ref, whn_ref,
              brz_ref, bin_ref, bhn_ref,
              wdyn_ref, bdyn_ref, out_ref):
    xb = x_ref[...]
    B = xb.shape[0]
    wip = wip_ref[...]
    bip = bip_ref[...]
    h = jnp.zeros((B, ENC), dtype=jnp.float32)
    for t in range(T):
        x0 = xb[:, 2 * t:2 * t + 1]
        x1 = xb[:, 2 * t + 1:2 * t + 2]
        emb = x0 * wip[0:1, :] + x1 * wip[1:2, :] + bip
        emb = jnp.maximum(emb, 0.1 * emb)
        r = jax.nn.sigmoid(jnp.dot(emb, wir_ref[...], preferred_element_type=jnp.float32)
                           + jnp.dot(h, whr_ref[...], preferred_element_type=jnp.float32)
                           + brz_ref[0:1, :])
        z = jax.nn.sigmoid(jnp.dot(emb, wiz_ref[...], preferred_element_type=jnp.float32)
                           + jnp.dot(h, whz_ref[...], preferred_element_type=jnp.float32)
                           + brz_ref[1:2, :])
        hn = jnp.dot(h, whn_ref[...], preferred_element_type=jnp.float32) + bhn_ref[...]
        xn = jnp.dot(emb, win_ref[...], preferred_element_type=jnp.float32) + bin_ref[...]
        n = jnp.tanh(xn + r * hn)
        h = (1.0 - z) * n + z * h
    out = jnp.dot(h, wdyn_ref[...], preferred_element_type=jnp.float32) + bdyn_ref[...]
    out_ref[...] = jnp.maximum(out, 0.1 * out)


def _encoder(x2d, p):
    grid = pl.cdiv(N, _BE)
    wi = p["gru_Wi"]
    wh = p["gru_Wh"]
    bi = p["gru_bi"]
    bh = p["gru_bh"]
    brz = jnp.stack([bi[0:ENC] + bh[0:ENC], bi[ENC:2 * ENC] + bh[ENC:2 * ENC]])
    full = lambda s: pl.BlockSpec(s, lambda i: tuple(0 for _ in s))
    return pl.pallas_call(
        _enc_body,
        grid=(grid,),
        in_specs=[
            pl.BlockSpec((_BE, T * D_IN), lambda i: (i, 0)),
            full((D_IN, EMB)), full((1, EMB)),
            full((EMB, ENC)), full((EMB, ENC)), full((EMB, ENC)),
            full((ENC, ENC)), full((ENC, ENC)), full((ENC, ENC)),
            full((2, ENC)), full((1, ENC)), full((1, ENC)),
            full((ENC, ENC)), full((1, ENC)),
        ],
        out_specs=pl.BlockSpec((_BE, ENC), lambda i: (i, 0)),
        out_shape=jax.ShapeDtypeStruct((N, ENC), jnp.float32),
    )(x2d, p["W_ip"], p["b_ip"][None, :],
      wi[:, 0:ENC], wi[:, ENC:2 * ENC], wi[:, 2 * ENC:],
      wh[:, 0:ENC], wh[:, ENC:2 * ENC], wh[:, 2 * ENC:],
      brz, bi[None, 2 * ENC:], bh[None, 2 * ENC:],
      p["W_dyn"], p["b_dyn"][None, :])


_BG = 1024


def _gat_dense_body(x_ref, w_ref, a_ref, xl_ref, sc_ref):
    xl = jnp.dot(x_ref[...], w_ref[...], preferred_element_type=jnp.float32)
    xl_ref[...] = xl
    sc_ref[...] = jnp.dot(xl, a_ref[...], preferred_element_type=jnp.float32)


def _gat_dense(x, w, a_src, a_dst):
    d_in = x.shape[1]
    amat = jnp.zeros((HC, 16), jnp.float32)
    for h in range(HEADS):
        amat = amat.at[h * ENC:(h + 1) * ENC, h].set(a_src[h])
        amat = amat.at[h * ENC:(h + 1) * ENC, h + 4].set(a_dst[h])
    grid = pl.cdiv(N, _BG)
    return pl.pallas_call(
        _gat_dense_body,
        grid=(grid,),
        in_specs=[
            pl.BlockSpec((_BG, d_in), lambda i: (i, 0)),
            pl.BlockSpec((d_in, HC), lambda i: (0, 0)),
            pl.BlockSpec((HC, 16), lambda i: (0, 0)),
        ],
        out_specs=[
            pl.BlockSpec((_BG, HC), lambda i: (i, 0)),
            pl.BlockSpec((_BG, 16), lambda i: (i, 0)),
        ],
        out_shape=[
            jax.ShapeDtypeStruct((N, HC), jnp.float32),
            jax.ShapeDtypeStruct((N, 16), jnp.float32),
        ],
    )(x, w, amat)


_EPT = 26624
_E2P = 32 * _EPT
_B1 = 1024
_K = 128
_PAD_SEG = 1 << 20


def _gat_edge_sc(xl, s16, src, dst, seg, nseg_real, C, nchunk):
    NV = nchunk * C
    C_CAP = C + 256
    ZR = C_CAP // 32
    R = C // 32
    i32 = jnp.int32
    f32 = jnp.float32

    def splits(total, step):
        out = []
        while total > 0:
            out.append(min(step, total))
            total -= out[-1]
        return out

    zsplits = splits(ZR, 32)
    fsplits = splits(R, 128)
    mesh = plsc.VectorSubcoreMesh(core_axis_name="c", subcore_axis_name="s")

    def body(src_hbm, dst_hbm, seg_hbm, xl_hbm, s16_hbm,
             num_hbm, den_hbm,
             srcb, dstb, segb, comp_pack, comp_seg,
             rows, s16s, s16d, wbuf, idxbuf, srcidx, dstidx, zrows, zden,
             numacc, denacc, sem):
        ci = lax.axis_index("c")
        si = lax.axis_index("s")
        wid = ci * 16 + si
        iota = lax.iota(i32, 16)
        zf = jnp.zeros((16,), f32)

        def zinit(r, _):
            rv = jnp.full((16,), r, i32)
            for kq in range(HC // 16):
                plsc.store_scatter(zrows, [rv, iota + 16 * kq], zf)
            plsc.store_scatter(zden, [rv, iota], zf)
            return 0
        lax.fori_loop(0, 32, zinit, 0)

        def winit(r, _):
            plsc.store_scatter(wbuf, [jnp.full((16,), r, i32), iota], zf)
            return 0
        lax.fori_loop(0, _K, winit, 0)

        def zero_acc():
            off = 0
            for n in zsplits:
                r0 = pl.multiple_of(wid * ZR + off, 8)
                pltpu.sync_copy(zrows.at[pl.ds(0, n)], numacc.at[pl.ds(r0, n)])
                pltpu.sync_copy(zden.at[pl.ds(0, n)], denacc.at[pl.ds(r0, n)])
                off += n

        zero_acc()
        plsc.subcore_barrier()

        def chunk(c, _):
            base = c * C
            hi = jnp.minimum(base + C, nseg_real)

            def stage(b, cnt):
                e0 = wid * _EPT + b * _B1
                pltpu.async_copy(seg_hbm.at[pl.ds(e0, _B1)], segb, sem).wait()
                pltpu.async_copy(src_hbm.at[pl.ds(e0, _B1)], srcb, sem).wait()
                pltpu.async_copy(dst_hbm.at[pl.ds(e0, _B1)], dstb, sem).wait()

                def vec(v, cnt):
                    o = pl.multiple_of(v * 16, 16)
                    segv = segb[pl.ds(o, 16)]
                    m = (segv >= base) & (segv < hi)
                    mi = m.astype(i32)
                    pos = cnt + plsc.cumsum(mi) - 1
                    pk = srcb[pl.ds(o, 16)] | lax.shift_left(dstb[pl.ds(o, 16)], 16)
                    plsc.store_scatter(comp_seg, [pos], segv - base, mask=m)
                    plsc.store_scatter(comp_pack, [pos], pk, mask=m)
                    return cnt + jnp.sum(mi)

                return lax.fori_loop(0, _B1 // 16, vec, cnt)

            cnt = lax.fori_loop(0, _EPT // _B1, stage, jnp.int32(0))

            zi = jnp.zeros((16,), i32)
            for k in range(_K // 16):
                posp = cnt + iota + 16 * k
                plsc.store_scatter(comp_seg, [posp], jnp.full((16,), C, i32))
                plsc.store_scatter(comp_pack, [posp], zi)

            def blk(j, _):
                o = pl.multiple_of(j * _K, _K)
                for k in range(_K // 16):
                    pv = comp_pack[pl.ds(o + 16 * k, 16)]
                    srcidx[pl.ds(16 * k, 16)] = pv & 0xFFFF
                    dstidx[pl.ds(16 * k, 16)] = lax.shift_right_logical(pv, 16)
                    idxbuf[pl.ds(16 * k, 16)] = comp_seg[pl.ds(o + 16 * k, 16)]
                pltpu.async_copy(xl_hbm.at[srcidx], rows, sem).wait()
                pltpu.async_copy(s16_hbm.at[srcidx], s16s, sem).wait()
                pltpu.async_copy(s16_hbm.at[dstidx], s16d, sem).wait()
                for g in range(_K // 16):
                    rv = iota + 16 * g
                    for h in range(HEADS):
                        ss = plsc.load_gather(s16s, [rv, jnp.full((16,), h, i32)])
                        sd = plsc.load_gather(s16d, [rv, jnp.full((16,), h + 4, i32)])
                        a = ss + sd
                        a = jnp.maximum(a, 0.2 * a)
                        plsc.store_scatter(wbuf, [rv, jnp.full((16,), h, i32)],
                                           jnp.exp(a))

                def scale(e, _):
                    ev = jnp.full((16,), e, i32)
                    for h in range(HEADS):
                        wb = plsc.load_gather(wbuf, [ev, jnp.full((16,), h, i32)])
                        for q in range(ENC // 16):
                            cv = iota + (ENC * h + 16 * q)
                            x = plsc.load_gather(rows, [ev, cv])
                            plsc.store_scatter(rows, [ev, cv], x * wb)
                    return 0
                lax.fori_loop(0, _K, scale, 0)

                pltpu.sync_copy(rows, numacc.at[idxbuf], add=True)
                pltpu.sync_copy(wbuf, denacc.at[idxbuf], add=True)
                return 0

            lax.fori_loop(0, (cnt + _K - 1) // _K, blk, 0)
            plsc.subcore_barrier()

            off = 0
            for n in fsplits:
                r0 = pl.multiple_of(wid * R + off, 8)
                pltpu.sync_copy(numacc.at[pl.ds(r0, n)],
                                num_hbm.at[ci, pl.ds(base + r0, n)])
                pltpu.sync_copy(denacc.at[pl.ds(r0, n)],
                                den_hbm.at[ci, pl.ds(base + r0, n)])
                off += n
            plsc.subcore_barrier()
            zero_acc()
            plsc.subcore_barrier()
            return 0

        lax.fori_loop(0, nchunk, chunk, 0)

    f = pl.kernel(
        body,
        out_type=[
            jax.ShapeDtypeStruct((2, NV, HC), f32),
            jax.ShapeDtypeStruct((2, NV, 16), f32),
        ],
        mesh=mesh,
        scratch_types=[
            pltpu.VMEM((_B1,), i32),
            pltpu.VMEM((_B1,), i32),
            pltpu.VMEM((_B1,), i32),
            pltpu.VMEM((_EPT + _K,), i32),
            pltpu.VMEM((_EPT + _K,), i32),
            pltpu.VMEM((_K, HC), f32),
            pltpu.VMEM((_K, 16), f32),
            pltpu.VMEM((_K, 16), f32),
            pltpu.VMEM((_K, 16), f32),
            pltpu.VMEM((_K,), i32),
            pltpu.VMEM((_K,), i32),
            pltpu.VMEM((_K,), i32),
            pltpu.VMEM((32, HC), f32),
            pltpu.VMEM((32, 16), f32),
            pltpu.VMEM_SHARED((C_CAP, HC), f32),
            pltpu.VMEM_SHARED((C_CAP, 16), f32),
            pltpu.SemaphoreType.DMA,
        ],
        compiler_params=pltpu.CompilerParams(use_tc_tiling_on_sc=False, needs_layout_passes=False),
    )
    return f(src, dst, seg, xl, s16)


_BC = 512


def _combine_body(num_ref, den_ref, bias_ref, out_ref):
    num = num_ref[0] + num_ref[1]
    den = den_ref[0] + den_ref[1]
    B = num.shape[0]
    db = jnp.concatenate(
        [jnp.broadcast_to(den[:, h:h + 1], (B, ENC)) for h in range(HEADS)],
        axis=1)
    out_ref[...] = num / (db + 1e-16) + bias_ref[...]


def _combine(num, den, bias, nout):
    NV = num.shape[1]
    nblk = pl.cdiv(nout, _BC)
    assert nblk * _BC <= NV
    out = pl.pallas_call(
        _combine_body,
        grid=(nblk,),
        in_specs=[
            pl.BlockSpec((2, _BC, HC), lambda i: (0, i, 0)),
            pl.BlockSpec((2, _BC, 16), lambda i: (0, i, 0)),
            pl.BlockSpec((1, HC), lambda i: (0, 0)),
        ],
        out_specs=pl.BlockSpec((_BC, HC), lambda i: (i, 0)),
        out_shape=jax.ShapeDtypeStruct((nblk * _BC, HC), jnp.float32),
    )(num, den, bias[None, :])
    return out[:nout]


def _gat_layer(x, srcp, dstp, segp, w, a_src, a_dst, bias, nseg_real, C,
               nchunk, nout):
    xl, s16 = _gat_dense(x, w, a_src, a_dst)
    num, den = _gat_edge_sc(xl, s16, srcp, dstp, segp, nseg_real, C, nchunk)
    return _combine(num, den, bias, nout)


_BD = 512


def _dec_body(xt_ref, wfc_ref, bfc_ref,
              wi0_ref, wh0_ref, b0_ref,
              wi1_ref, wh1_ref, b1_ref,
              wop_ref, bop_ref, out_ref):
    B = xt_ref.shape[0]
    enc = jnp.dot(xt_ref[...], wfc_ref[...], preferred_element_type=jnp.float32) + bfc_ref[...]
    enc = jnp.maximum(enc, 0.1 * enc)
    gx0 = jnp.dot(enc, wi0_ref[...], preferred_element_type=jnp.float32) + b0_ref[...]
    h1 = jnp.zeros((B, DEC), jnp.float32)
    c1 = jnp.zeros((B, DEC), jnp.float32)
    h2 = jnp.zeros((B, DEC), jnp.float32)
    c2 = jnp.zeros((B, DEC), jnp.float32)
    wop = wop_ref[...]
    bop = bop_ref[...]
    for t in range(OUT_LEN):
        g = gx0 + jnp.dot(h1, wh0_ref[...], preferred_element_type=jnp.float32)
        i = jax.nn.sigmoid(g[:, 0:DEC])
        f = jax.nn.sigmoid(g[:, DEC:2 * DEC])
        gg = jnp.tanh(g[:, 2 * DEC:3 * DEC])
        o = jax.nn.sigmoid(g[:, 3 * DEC:])
        c1 = f * c1 + i * gg
        h1 = o * jnp.tanh(c1)
        g = (jnp.dot(h1, wi1_ref[...], preferred_element_type=jnp.float32)
             + jnp.dot(h2, wh1_ref[...], preferred_element_type=jnp.float32) + b1_ref[...])
        i = jax.nn.sigmoid(g[:, 0:DEC])
        f = jax.nn.sigmoid(g[:, DEC:2 * DEC])
        gg = jnp.tanh(g[:, 2 * DEC:3 * DEC])
        o = jax.nn.sigmoid(g[:, 3 * DEC:])
        c2 = f * c2 + i * gg
        h2 = o * jnp.tanh(c2)
        out_ref[:, pl.ds(2 * t, 2)] = (
            jnp.dot(h2, wop, preferred_element_type=jnp.float32) + bop)


def _decoder(g2t, p):
    grid = pl.cdiv(G, _BD)
    full = lambda s: pl.BlockSpec(s, lambda i: tuple(0 for _ in s))
    out = pl.pallas_call(
        _dec_body,
        grid=(grid,),
        in_specs=[
            pl.BlockSpec((_BD, HC), lambda i: (i, 0)),
            full((HC, ENC)), full((1, ENC)),
            full((ENC, 4 * DEC)), full((DEC, 4 * DEC)), full((1, 4 * DEC)),
            full((DEC, 4 * DEC)), full((DEC, 4 * DEC)), full((1, 4 * DEC)),
            full((DEC, 2)), full((1, 2)),
        ],
        out_specs=pl.BlockSpec((_BD, 2 * OUT_LEN), lambda i: (i, 0)),
        out_shape=jax.ShapeDtypeStruct((G, 2 * OUT_LEN), jnp.float32),
    )(g2t, p["W_fc"], p["b_fc"][None, :],
      p["lstm0_Wi"], p["lstm0_Wh"], p["lstm0_b"][None, :],
      p["lstm1_Wi"], p["lstm1_Wh"], p["lstm1_b"][None, :],
      p["W_op"], p["b_op"][None, :])
    return out.reshape(G, OUT_LEN, 2)


def kernel(x, params, edge_index, batch, num_graphs):
    p = params
    x2d = x.reshape(N, T * D_IN)
    henc = _encoder(x2d, p)

    ei = edge_index.astype(jnp.int32)
    loop = jnp.arange(N, dtype=jnp.int32)
    src = jnp.concatenate([ei[0], loop])
    dst = jnp.concatenate([ei[1], loop])
    npad = _E2P - src.shape[0]
    srcp = jnp.pad(src, (0, npad))
    dstp = jnp.pad(dst, (0, npad))
    segp1 = jnp.pad(dst, (0, npad), constant_values=_PAD_SEG)

    g1 = _gat_layer(henc, srcp, dstp, segp1, p["gat1_W"], p["gat1_asrc"],
                    p["gat1_adst"], p["gat1_b"], nseg_real=N, C=2048,
                    nchunk=25, nout=N)

    b32 = batch.astype(jnp.int32)
    is_tgt = jnp.concatenate([jnp.ones((1,), jnp.bool_), b32[1:] != b32[:-1]])
    slot = jnp.where(is_tgt, b32, _PAD_SEG)
    seg2 = jnp.pad(slot[dst], (0, npad), constant_values=_PAD_SEG)
    g2t = _gat_layer(g1, srcp, dstp, seg2, p["gat2_W"], p["gat2_asrc"],
                     p["gat2_adst"], p["gat2_b"], nseg_real=G, C=2048,
                     nchunk=1, nout=G)
    return _decoder(g2t, p)


# speedup vs baseline: 5.5297x; 1.2271x over previous
import functools

import jax
import jax.numpy as jnp
from jax import lax
from jax.experimental import pallas as pl
from jax.experimental.pallas import tpu as pltpu
from jax.experimental.pallas import tpu_sc as plsc

N = 50000
T = 16
D_IN = 2
EMB = 32
ENC = 64
HEADS = 3
DEC = 128
OUT_LEN = 25
HC = HEADS * ENC
G = 2048


_BE = 512


def _enc_body(x_ref, wip_ref, bip_ref,
              wir_ref, wiz_ref, win_ref,
              whr_ref, whz_ref, whn_ref,
              brz_ref, bin_ref, bhn_ref,
              wdyn_ref, bdyn_ref, out_ref):
    xb = x_ref[...]
    B = xb.shape[0]
    wip = wip_ref[...]
    bip = bip_ref[...]
    h = jnp.zeros((B, ENC), dtype=jnp.float32)
    for t in range(T):
        x0 = xb[:, 2 * t:2 * t + 1]
        x1 = xb[:, 2 * t + 1:2 * t + 2]
        emb = x0 * wip[0:1, :] + x1 * wip[1:2, :] + bip
        emb = jnp.maximum(emb, 0.1 * emb)
        r = jax.nn.sigmoid(jnp.dot(emb, wir_ref[...], preferred_element_type=jnp.float32)
                           + jnp.dot(h, whr_ref[...], preferred_element_type=jnp.float32)
                           + brz_ref[0:1, :])
        z = jax.nn.sigmoid(jnp.dot(emb, wiz_ref[...], preferred_element_type=jnp.float32)
                           + jnp.dot(h, whz_ref[...], preferred_element_type=jnp.float32)
                           + brz_ref[1:2, :])
        hn = jnp.dot(h, whn_ref[...], preferred_element_type=jnp.float32) + bhn_ref[...]
        xn = jnp.dot(emb, win_ref[...], preferred_element_type=jnp.float32) + bin_ref[...]
        n = jnp.tanh(xn + r * hn)
        h = (1.0 - z) * n + z * h
    out = jnp.dot(h, wdyn_ref[...], preferred_element_type=jnp.float32) + bdyn_ref[...]
    out_ref[...] = jnp.maximum(out, 0.1 * out)


def _encoder(x2d, p):
    grid = pl.cdiv(N, _BE)
    wi = p["gru_Wi"]
    wh = p["gru_Wh"]
    bi = p["gru_bi"]
    bh = p["gru_bh"]
    brz = jnp.stack([bi[0:ENC] + bh[0:ENC], bi[ENC:2 * ENC] + bh[ENC:2 * ENC]])
    full = lambda s: pl.BlockSpec(s, lambda i: tuple(0 for _ in s))
    return pl.pallas_call(
        _enc_body,
        grid=(grid,),
        in_specs=[
            pl.BlockSpec((_BE, T * D_IN), lambda i: (i, 0)),
            full((D_IN, EMB)), full((1, EMB)),
            full((EMB, ENC)), full((EMB, ENC)), full((EMB, ENC)),
            full((ENC, ENC)), full((ENC, ENC)), full((ENC, ENC)),
            full((2, ENC)), full((1, ENC)), full((1, ENC)),
            full((ENC, ENC)), full((1, ENC)),
        ],
        out_specs=pl.BlockSpec((_BE, ENC), lambda i: (i, 0)),
        out_shape=jax.ShapeDtypeStruct((N, ENC), jnp.float32),
    )(x2d, p["W_ip"], p["b_ip"][None, :],
      wi[:, 0:ENC], wi[:, ENC:2 * ENC], wi[:, 2 * ENC:],
      wh[:, 0:ENC], wh[:, ENC:2 * ENC], wh[:, 2 * ENC:],
      brz, bi[None, 2 * ENC:], bh[None, 2 * ENC:],
      p["W_dyn"], p["b_dyn"][None, :])


_BG = 1024


def _gat_dense_body(x_ref, w_ref, a_ref, xl_ref, sc_ref):
    xl = jnp.dot(x_ref[...], w_ref[...], preferred_element_type=jnp.float32)
    xl_ref[...] = xl
    sc_ref[...] = jnp.dot(xl, a_ref[...], preferred_element_type=jnp.float32)


def _gat_dense(x, w, a_src, a_dst):
    d_in = x.shape[1]
    amat = jnp.zeros((HC, 16), jnp.float32)
    for h in range(HEADS):
        amat = amat.at[h * ENC:(h + 1) * ENC, h].set(a_src[h])
        amat = amat.at[h * ENC:(h + 1) * ENC, h + 4].set(a_dst[h])
    grid = pl.cdiv(N, _BG)
    return pl.pallas_call(
        _gat_dense_body,
        grid=(grid,),
        in_specs=[
            pl.BlockSpec((_BG, d_in), lambda i: (i, 0)),
            pl.BlockSpec((d_in, HC), lambda i: (0, 0)),
            pl.BlockSpec((HC, 16), lambda i: (0, 0)),
        ],
        out_specs=[
            pl.BlockSpec((_BG, HC), lambda i: (i, 0)),
            pl.BlockSpec((_BG, 16), lambda i: (i, 0)),
        ],
        out_shape=[
            jax.ShapeDtypeStruct((N, HC), jnp.float32),
            jax.ShapeDtypeStruct((N, 16), jnp.float32),
        ],
    )(x, w, amat)


_EPT = 26624
_E2P = 32 * _EPT
_K = 128
_PAD_SEG = 1 << 20


def _gat_edge_sc(xl, s16, pack, seg, nseg_real, C, nchunk):
    NV = nchunk * C
    C_CAP = C + 256
    ZR = C_CAP // 32
    R = C // 32
    i32 = jnp.int32
    f32 = jnp.float32

    def splits(total, step):
        out = []
        while total > 0:
            out.append(min(step, total))
            total -= out[-1]
        return out

    zsplits = splits(ZR, 32)
    fsplits = splits(R, 128)
    mesh = plsc.VectorSubcoreMesh(core_axis_name="c", subcore_axis_name="s")

    def body(pack_hbm, seg_hbm, xl_hbm, s16_hbm,
             num_hbm, den_hbm,
             seg_all, comp_idx, packb,
             rows, s16s, s16d, wbuf, idxbuf, gidx, srcidx, dstidx,
             zrows, zden,
             numacc, denacc, sem):
        ci = lax.axis_index("c")
        si = lax.axis_index("s")
        wid = ci * 16 + si
        iota = lax.iota(i32, 16)
        zf = jnp.zeros((16,), f32)

        pltpu.async_copy(seg_hbm.at[pl.ds(wid * _EPT, _EPT)],
                         seg_all.at[pl.ds(0, _EPT)], sem).wait()

        def zinit(r, _):
            rv = jnp.full((16,), r, i32)
            for kq in range(HC // 16):
                plsc.store_scatter(zrows, [rv, iota + 16 * kq], zf)
            plsc.store_scatter(zden, [rv, iota], zf)
            return 0
        lax.fori_loop(0, 32, zinit, 0)

        def winit(r, _):
            plsc.store_scatter(wbuf, [jnp.full((16,), r, i32), iota], zf)
            return 0
        lax.fori_loop(0, _K, winit, 0)

        def zero_acc():
            off = 0
            for n in zsplits:
                r0 = pl.multiple_of(wid * ZR + off, 8)
                pltpu.sync_copy(zrows.at[pl.ds(0, n)], numacc.at[pl.ds(r0, n)])
                pltpu.sync_copy(zden.at[pl.ds(0, n)], denacc.at[pl.ds(r0, n)])
                off += n

        zero_acc()
        plsc.subcore_barrier()

        def chunk(c, _):
            base = c * C
            hi = jnp.minimum(base + C, nseg_real)
            plsc.store_scatter(seg_all, [iota + _EPT],
                               jnp.full((16,), base + C, i32))

            def vec(v, cnt):
                o = pl.multiple_of(v * 16, 16)
                segv = seg_all[pl.ds(o, 16)]
                m = (segv >= base) & (segv < hi)
                mi = m.astype(i32)
                pos = cnt + plsc.cumsum(mi) - 1
                plsc.store_scatter(comp_idx, [pos], iota + o, mask=m)
                return cnt + jnp.sum(mi)

            cnt = lax.fori_loop(0, _EPT // 16, vec, jnp.int32(0))

            for k in range(_K // 16):
                plsc.store_scatter(comp_idx, [cnt + iota + 16 * k],
                                   jnp.full((16,), _EPT, i32))

            def blk(j, _):
                o = pl.multiple_of(j * _K, _K)
                for k in range(_K // 16):
                    lv = comp_idx[pl.ds(o + 16 * k, 16)]
                    segv = plsc.load_gather(seg_all, [lv])
                    idxbuf[pl.ds(16 * k, 16)] = segv - base
                    gidx[pl.ds(16 * k, 16)] = lv + wid * _EPT
                pltpu.async_copy(pack_hbm.at[gidx], packb, sem).wait()
                for k in range(_K // 16):
                    pv = packb[pl.ds(16 * k, 16)]
                    srcidx[pl.ds(16 * k, 16)] = pv & 0xFFFF
                    dstidx[pl.ds(16 * k, 16)] = lax.shift_right_logical(pv, 16)
                c1 = pltpu.async_copy(xl_hbm.at[srcidx], rows, sem)
                c2 = pltpu.async_copy(s16_hbm.at[srcidx], s16s, sem)
                c3 = pltpu.async_copy(s16_hbm.at[dstidx], s16d, sem)
                c1.wait()
                c2.wait()
                c3.wait()
                for g in range(_K // 16):
                    rv = iota + 16 * g
                    for h in range(HEADS):
                        ss = plsc.load_gather(s16s, [rv, jnp.full((16,), h, i32)])
                        sd = plsc.load_gather(s16d, [rv, jnp.full((16,), h + 4, i32)])
                        a = ss + sd
                        a = jnp.maximum(a, 0.2 * a)
                        plsc.store_scatter(wbuf, [rv, jnp.full((16,), h, i32)],
                                           jnp.exp(a))

                def scale(e, _):
                    ev = jnp.full((16,), e, i32)
                    for h in range(HEADS):
                        wb = plsc.load_gather(wbuf, [ev, jnp.full((16,), h, i32)])
                        for q in range(ENC // 16):
                            cv = iota + (ENC * h + 16 * q)
                            x = plsc.load_gather(rows, [ev, cv])
                            plsc.store_scatter(rows, [ev, cv], x * wb)
                    return 0
                lax.fori_loop(0, _K, scale, 0)

                pltpu.sync_copy(rows, numacc.at[idxbuf], add=True)
                pltpu.sync_copy(wbuf, denacc.at[idxbuf], add=True)
                return 0

            lax.fori_loop(0, (cnt + _K - 1) // _K, blk, 0)
            plsc.subcore_barrier()

            off = 0
            for n in fsplits:
                r0 = pl.multiple_of(wid * R + off, 8)
                pltpu.sync_copy(numacc.at[pl.ds(r0, n)],
                                num_hbm.at[ci, pl.ds(base + r0, n)])
                pltpu.sync_copy(denacc.at[pl.ds(r0, n)],
                                den_hbm.at[ci, pl.ds(base + r0, n)])
                off += n
            plsc.subcore_barrier()
            zero_acc()
            plsc.subcore_barrier()
            return 0

        lax.fori_loop(0, nchunk, chunk, 0)

    f = pl.kernel(
        body,
        out_type=[
            jax.ShapeDtypeStruct((2, NV, HC), f32),
            jax.ShapeDtypeStruct((2, NV, 16), f32),
        ],
        mesh=mesh,
        scratch_types=[
            pltpu.VMEM((_EPT + 16,), i32),
            pltpu.VMEM((_EPT + _K,), i32),
            pltpu.VMEM((_K,), i32),
            pltpu.VMEM((_K, HC), f32),
            pltpu.VMEM((_K, 16), f32),
            pltpu.VMEM((_K, 16), f32),
            pltpu.VMEM((_K, 16), f32),
            pltpu.VMEM((_K,), i32),
            pltpu.VMEM((_K,), i32),
            pltpu.VMEM((_K,), i32),
            pltpu.VMEM((_K,), i32),
            pltpu.VMEM((32, HC), f32),
            pltpu.VMEM((32, 16), f32),
            pltpu.VMEM_SHARED((C_CAP, HC), f32),
            pltpu.VMEM_SHARED((C_CAP, 16), f32),
            pltpu.SemaphoreType.DMA,
        ],
        compiler_params=pltpu.CompilerParams(use_tc_tiling_on_sc=False, needs_layout_passes=False),
    )
    return f(pack, seg, xl, s16)


_BC = 512


def _combine_body(num_ref, den_ref, bias_ref, out_ref):
    num = num_ref[0] + num_ref[1]
    den = den_ref[0] + den_ref[1]
    B = num.shape[0]
    db = jnp.concatenate(
        [jnp.broadcast_to(den[:, h:h + 1], (B, ENC)) for h in range(HEADS)],
        axis=1)
    out_ref[...] = num / (db + 1e-16) + bias_ref[...]


def _combine(num, den, bias, nout):
    NV = num.shape[1]
    nblk = pl.cdiv(nout, _BC)
    assert nblk * _BC <= NV
    out = pl.pallas_call(
        _combine_body,
        grid=(nblk,),
        in_specs=[
            pl.BlockSpec((2, _BC, HC), lambda i: (0, i, 0)),
            pl.BlockSpec((2, _BC, 16), lambda i: (0, i, 0)),
            pl.BlockSpec((1, HC), lambda i: (0, 0)),
        ],
        out_specs=pl.BlockSpec((_BC, HC), lambda i: (i, 0)),
        out_shape=jax.ShapeDtypeStruct((nblk * _BC, HC), jnp.float32),
    )(num, den, bias[None, :])
    return out[:nout]


def _gat_layer(x, packp, segp, w, a_src, a_dst, bias, nseg_real, C,
               nchunk, nout):
    xl, s16 = _gat_dense(x, w, a_src, a_dst)
    num, den = _gat_edge_sc(xl, s16, packp, segp, nseg_real, C, nchunk)
    return _combine(num, den, bias, nout)


_BD = 512


def _dec_body(xt_ref, wfc_ref, bfc_ref,
              wi0_ref, wh0_ref, b0_ref,
              wi1_ref, wh1_ref, b1_ref,
              wop_ref, bop_ref, out_ref):
    B = xt_ref.shape[0]
    enc = jnp.dot(xt_ref[...], wfc_ref[...], preferred_element_type=jnp.float32) + bfc_ref[...]
    enc = jnp.maximum(enc, 0.1 * enc)
    gx0 = jnp.dot(enc, wi0_ref[...], preferred_element_type=jnp.float32) + b0_ref[...]
    h1 = jnp.zeros((B, DEC), jnp.float32)
    c1 = jnp.zeros((B, DEC), jnp.float32)
    h2 = jnp.zeros((B, DEC), jnp.float32)
    c2 = jnp.zeros((B, DEC), jnp.float32)
    wop = wop_ref[...]
    bop = bop_ref[...]
    for t in range(OUT_LEN):
        g = gx0 + jnp.dot(h1, wh0_ref[...], preferred_element_type=jnp.float32)
        i = jax.nn.sigmoid(g[:, 0:DEC])
        f = jax.nn.sigmoid(g[:, DEC:2 * DEC])
        gg = jnp.tanh(g[:, 2 * DEC:3 * DEC])
        o = jax.nn.sigmoid(g[:, 3 * DEC:])
        c1 = f * c1 + i * gg
        h1 = o * jnp.tanh(c1)
        g = (jnp.dot(h1, wi1_ref[...], preferred_element_type=jnp.float32)
             + jnp.dot(h2, wh1_ref[...], preferred_element_type=jnp.float32) + b1_ref[...])
        i = jax.nn.sigmoid(g[:, 0:DEC])
        f = jax.nn.sigmoid(g[:, DEC:2 * DEC])
        gg = jnp.tanh(g[:, 2 * DEC:3 * DEC])
        o = jax.nn.sigmoid(g[:, 3 * DEC:])
        c2 = f * c2 + i * gg
        h2 = o * jnp.tanh(c2)
        out_ref[:, pl.ds(2 * t, 2)] = (
            jnp.dot(h2, wop, preferred_element_type=jnp.float32) + bop)


def _decoder(g2t, p):
    grid = pl.cdiv(G, _BD)
    full = lambda s: pl.BlockSpec(s, lambda i: tuple(0 for _ in s))
    out = pl.pallas_call(
        _dec_body,
        grid=(grid,),
        in_specs=[
            pl.BlockSpec((_BD, HC), lambda i: (i, 0)),
            full((HC, ENC)), full((1, ENC)),
            full((ENC, 4 * DEC)), full((DEC, 4 * DEC)), full((1, 4 * DEC)),
            full((DEC, 4 * DEC)), full((DEC, 4 * DEC)), full((1, 4 * DEC)),
            full((DEC, 2)), full((1, 2)),
        ],
        out_specs=pl.BlockSpec((_BD, 2 * OUT_LEN), lambda i: (i, 0)),
        out_shape=jax.ShapeDtypeStruct((G, 2 * OUT_LEN), jnp.float32),
    )(g2t, p["W_fc"], p["b_fc"][None, :],
      p["lstm0_Wi"], p["lstm0_Wh"], p["lstm0_b"][None, :],
      p["lstm1_Wi"], p["lstm1_Wh"], p["lstm1_b"][None, :],
      p["W_op"], p["b_op"][None, :])
    return out.reshape(G, OUT_LEN, 2)


def kernel(x, params, edge_index, batch, num_graphs):
    p = params
    x2d = x.reshape(N, T * D_IN)
    henc = _encoder(x2d, p)

    ei = edge_index.astype(jnp.int32)
    loop = jnp.arange(N, dtype=jnp.int32)
    src = jnp.concatenate([ei[0], loop])
    dst = jnp.concatenate([ei[1], loop])
    npad = _E2P - src.shape[0]
    packp = jnp.pad(src | (dst << 16), (0, npad + 16))
    segp1 = jnp.pad(dst, (0, npad), constant_values=_PAD_SEG)

    g1 = _gat_layer(henc, packp, segp1, p["gat1_W"], p["gat1_asrc"],
                    p["gat1_adst"], p["gat1_b"], nseg_real=N, C=2048,
                    nchunk=25, nout=N)

    b32 = batch.astype(jnp.int32)
    is_tgt = jnp.concatenate([jnp.ones((1,), jnp.bool_), b32[1:] != b32[:-1]])
    slot = jnp.where(is_tgt, b32, _PAD_SEG)
    seg2 = jnp.pad(slot[dst], (0, npad), constant_values=_PAD_SEG)
    g2t = _gat_layer(g1, packp, seg2, p["gat2_W"], p["gat2_asrc"],
                     p["gat2_adst"], p["gat2_b"], nseg_real=G, C=2048,
                     nchunk=1, nout=G)
    return _decoder(g2t, p)


# speedup vs baseline: 5.7696x; 1.0434x over previous
import functools

import jax
import jax.numpy as jnp
from jax import lax
from jax.experimental import pallas as pl
from jax.experimental.pallas import tpu as pltpu
from jax.experimental.pallas import tpu_sc as plsc

N = 50000
T = 16
D_IN = 2
EMB = 32
ENC = 64
HEADS = 3
DEC = 128
OUT_LEN = 25
HC = HEADS * ENC
G = 2048


_BE = 512


def _enc_body(x_ref, wip_ref, bip_ref,
              wir_ref, wiz_ref, win_ref,
              whr_ref, whz_ref, whn_ref,
              brz_ref, bin_ref, bhn_ref,
              wdyn_ref, bdyn_ref, out_ref):
    xb = x_ref[...]
    B = xb.shape[0]
    wip = wip_ref[...]
    bip = bip_ref[...]
    h = jnp.zeros((B, ENC), dtype=jnp.float32)
    for t in range(T):
        x0 = xb[:, 2 * t:2 * t + 1]
        x1 = xb[:, 2 * t + 1:2 * t + 2]
        emb = x0 * wip[0:1, :] + x1 * wip[1:2, :] + bip
        emb = jnp.maximum(emb, 0.1 * emb)
        r = jax.nn.sigmoid(jnp.dot(emb, wir_ref[...], preferred_element_type=jnp.float32)
                           + jnp.dot(h, whr_ref[...], preferred_element_type=jnp.float32)
                           + brz_ref[0:1, :])
        z = jax.nn.sigmoid(jnp.dot(emb, wiz_ref[...], preferred_element_type=jnp.float32)
                           + jnp.dot(h, whz_ref[...], preferred_element_type=jnp.float32)
                           + brz_ref[1:2, :])
        hn = jnp.dot(h, whn_ref[...], preferred_element_type=jnp.float32) + bhn_ref[...]
        xn = jnp.dot(emb, win_ref[...], preferred_element_type=jnp.float32) + bin_ref[...]
        n = jnp.tanh(xn + r * hn)
        h = (1.0 - z) * n + z * h
    out = jnp.dot(h, wdyn_ref[...], preferred_element_type=jnp.float32) + bdyn_ref[...]
    out_ref[...] = jnp.maximum(out, 0.1 * out)


def _encoder(x2d, p):
    grid = pl.cdiv(N, _BE)
    wi = p["gru_Wi"]
    wh = p["gru_Wh"]
    bi = p["gru_bi"]
    bh = p["gru_bh"]
    brz = jnp.stack([bi[0:ENC] + bh[0:ENC], bi[ENC:2 * ENC] + bh[ENC:2 * ENC]])
    full = lambda s: pl.BlockSpec(s, lambda i: tuple(0 for _ in s))
    return pl.pallas_call(
        _enc_body,
        grid=(grid,),
        in_specs=[
            pl.BlockSpec((_BE, T * D_IN), lambda i: (i, 0)),
            full((D_IN, EMB)), full((1, EMB)),
            full((EMB, ENC)), full((EMB, ENC)), full((EMB, ENC)),
            full((ENC, ENC)), full((ENC, ENC)), full((ENC, ENC)),
            full((2, ENC)), full((1, ENC)), full((1, ENC)),
            full((ENC, ENC)), full((1, ENC)),
        ],
        out_specs=pl.BlockSpec((_BE, ENC), lambda i: (i, 0)),
        out_shape=jax.ShapeDtypeStruct((N, ENC), jnp.float32),
    )(x2d, p["W_ip"], p["b_ip"][None, :],
      wi[:, 0:ENC], wi[:, ENC:2 * ENC], wi[:, 2 * ENC:],
      wh[:, 0:ENC], wh[:, ENC:2 * ENC], wh[:, 2 * ENC:],
      brz, bi[None, 2 * ENC:], bh[None, 2 * ENC:],
      p["W_dyn"], p["b_dyn"][None, :])


_BG = 1024


def _gat_dense_body(x_ref, w_ref, a_ref, xl_ref, sc_ref):
    xl = jnp.dot(x_ref[...], w_ref[...], preferred_element_type=jnp.float32)
    xl_ref[...] = xl
    sc_ref[...] = jnp.dot(xl, a_ref[...], preferred_element_type=jnp.float32)


def _gat_dense(x, w, a_src, a_dst):
    d_in = x.shape[1]
    amat = jnp.zeros((HC, 16), jnp.float32)
    for h in range(HEADS):
        amat = amat.at[h * ENC:(h + 1) * ENC, h].set(a_src[h])
        amat = amat.at[h * ENC:(h + 1) * ENC, h + 4].set(a_dst[h])
    grid = pl.cdiv(N, _BG)
    return pl.pallas_call(
        _gat_dense_body,
        grid=(grid,),
        in_specs=[
            pl.BlockSpec((_BG, d_in), lambda i: (i, 0)),
            pl.BlockSpec((d_in, HC), lambda i: (0, 0)),
            pl.BlockSpec((HC, 16), lambda i: (0, 0)),
        ],
        out_specs=[
            pl.BlockSpec((_BG, HC), lambda i: (i, 0)),
            pl.BlockSpec((_BG, 16), lambda i: (i, 0)),
        ],
        out_shape=[
            jax.ShapeDtypeStruct((N, HC), jnp.float32),
            jax.ShapeDtypeStruct((N, 16), jnp.float32),
        ],
    )(x, w, amat)


_EPT = 26624
_E2P = 32 * _EPT
_K = 64
_PAD_SEG = 1 << 20


def _gat_edge_sc(xl, s16, pack, seg, nseg_real, C, nchunk):
    NV = nchunk * C
    C_CAP = C + 256
    ZR = C_CAP // 32
    R = C // 32
    i32 = jnp.int32
    f32 = jnp.float32

    def splits(total, step):
        out = []
        while total > 0:
            out.append(min(step, total))
            total -= out[-1]
        return out

    zsplits = splits(ZR, 32)
    fsplits = splits(R, 128)
    mesh = plsc.VectorSubcoreMesh(core_axis_name="c", subcore_axis_name="s")

    def body(pack_hbm, seg_hbm, xl_hbm, s16_hbm,
             num_hbm, den_hbm,
             seg_all, comp_idx, pk_a, pk_b,
             rows_a, rows_b, sxs_a, sxs_b, sxd_a, sxd_b, wbuf,
             idx_a, idx_b, gix_a, gix_b, si_a, si_b, di_a, di_b,
             zrows, zden,
             numacc, denacc, sem):
        ci = lax.axis_index("c")
        si = lax.axis_index("s")
        wid = ci * 16 + si
        iota = lax.iota(i32, 16)
        zf = jnp.zeros((16,), f32)

        pltpu.async_copy(seg_hbm.at[pl.ds(wid * _EPT, _EPT)],
                         seg_all.at[pl.ds(0, _EPT)], sem).wait()

        def zinit(r, _):
            rv = jnp.full((16,), r, i32)
            for kq in range(HC // 16):
                plsc.store_scatter(zrows, [rv, iota + 16 * kq], zf)
            plsc.store_scatter(zden, [rv, iota], zf)
            return 0
        lax.fori_loop(0, 32, zinit, 0)

        def winit(r, _):
            plsc.store_scatter(wbuf, [jnp.full((16,), r, i32), iota], zf)
            return 0
        lax.fori_loop(0, _K, winit, 0)

        def zero_acc():
            off = 0
            for n in zsplits:
                r0 = pl.multiple_of(wid * ZR + off, 8)
                pltpu.sync_copy(zrows.at[pl.ds(0, n)], numacc.at[pl.ds(r0, n)])
                pltpu.sync_copy(zden.at[pl.ds(0, n)], denacc.at[pl.ds(r0, n)])
                off += n

        zero_acc()
        plsc.subcore_barrier()

        def chunk(c, _):
            base = c * C
            hi = jnp.minimum(base + C, nseg_real)
            plsc.store_scatter(seg_all, [iota + _EPT],
                               jnp.full((16,), base + C, i32))

            def vec(v, cnt):
                o = pl.multiple_of(v * 16, 16)
                segv = seg_all[pl.ds(o, 16)]
                m = (segv >= base) & (segv < hi)
                mi = m.astype(i32)
                pos = cnt + plsc.cumsum(mi) - 1
                plsc.store_scatter(comp_idx, [pos], iota + o, mask=m)
                return cnt + jnp.sum(mi)

            cnt = lax.fori_loop(0, _EPT // 16, vec, jnp.int32(0))

            for k in range(2 * _K // 16):
                plsc.store_scatter(comp_idx, [cnt + iota + 16 * k],
                                   jnp.full((16,), _EPT, i32))

            def prep(o, idx_r, gix_r, pk_r):
                for k in range(_K // 16):
                    lv = comp_idx[pl.ds(o + 16 * k, 16)]
                    segv = plsc.load_gather(seg_all, [lv])
                    idx_r[pl.ds(16 * k, 16)] = segv - base
                    gix_r[pl.ds(16 * k, 16)] = lv + wid * _EPT
                return pltpu.async_copy(pack_hbm.at[gix_r], pk_r, sem)

            def row_gather(pk_r, si_r, di_r, rows_r, ss_r, sd_r):
                for k in range(_K // 16):
                    pv = pk_r[pl.ds(16 * k, 16)]
                    si_r[pl.ds(16 * k, 16)] = pv & 0xFFFF
                    di_r[pl.ds(16 * k, 16)] = lax.shift_right_logical(pv, 16)
                c1 = pltpu.async_copy(xl_hbm.at[si_r], rows_r, sem)
                c2 = pltpu.async_copy(s16_hbm.at[si_r], ss_r, sem)
                c3 = pltpu.async_copy(s16_hbm.at[di_r], sd_r, sem)
                return c1, c2, c3

            def compute(rows_r, ss_r, sd_r, idx_r):
                for g in range(_K // 16):
                    rv = iota + 16 * g
                    for h in range(HEADS):
                        ss = plsc.load_gather(ss_r, [rv, jnp.full((16,), h, i32)])
                        sd = plsc.load_gather(sd_r, [rv, jnp.full((16,), h + 4, i32)])
                        a = ss + sd
                        a = jnp.maximum(a, 0.2 * a)
                        plsc.store_scatter(wbuf, [rv, jnp.full((16,), h, i32)],
                                           jnp.exp(a))

                def scale(e, _):
                    ev = jnp.full((16,), e, i32)
                    for h in range(HEADS):
                        wb = plsc.load_gather(wbuf, [ev, jnp.full((16,), h, i32)])
                        for q in range(ENC // 16):
                            cv = iota + (ENC * h + 16 * q)
                            x = plsc.load_gather(rows_r, [ev, cv])
                            plsc.store_scatter(rows_r, [ev, cv], x * wb)
                    return 0
                lax.fori_loop(0, _K, scale, 0)

                pltpu.sync_copy(rows_r, numacc.at[idx_r], add=True)
                pltpu.sync_copy(wbuf, denacc.at[idx_r], add=True)

            def pair(p, _):
                o0 = pl.multiple_of(p * 2 * _K, _K)
                o1 = o0 + _K
                cp0 = prep(o0, idx_a, gix_a, pk_a)
                cp0.wait()
                g0 = row_gather(pk_a, si_a, di_a, rows_a, sxs_a, sxd_a)
                cp1 = prep(o1, idx_b, gix_b, pk_b)
                for h in g0:
                    h.wait()
                cp1.wait()
                g1 = row_gather(pk_b, si_b, di_b, rows_b, sxs_b, sxd_b)
                compute(rows_a, sxs_a, sxd_a, idx_a)
                for h in g1:
                    h.wait()
                compute(rows_b, sxs_b, sxd_b, idx_b)
                return 0

            lax.fori_loop(0, (cnt + 2 * _K - 1) // (2 * _K), pair, 0)
            plsc.subcore_barrier()

            off = 0
            for n in fsplits:
                r0 = pl.multiple_of(wid * R + off, 8)
                pltpu.sync_copy(numacc.at[pl.ds(r0, n)],
                                num_hbm.at[ci, pl.ds(base + r0, n)])
                pltpu.sync_copy(denacc.at[pl.ds(r0, n)],
                                den_hbm.at[ci, pl.ds(base + r0, n)])
                off += n
            plsc.subcore_barrier()
            zero_acc()
            plsc.subcore_barrier()
            return 0

        lax.fori_loop(0, nchunk, chunk, 0)

    f = pl.kernel(
        body,
        out_type=[
            jax.ShapeDtypeStruct((2, NV, HC), f32),
            jax.ShapeDtypeStruct((2, NV, 16), f32),
        ],
        mesh=mesh,
        scratch_types=(
            [pltpu.VMEM((_EPT + 16,), i32),
             pltpu.VMEM((_EPT + 2 * _K,), i32)]
            + [pltpu.VMEM((_K,), i32)] * 2
            + [pltpu.VMEM((_K, HC), f32)] * 2
            + [pltpu.VMEM((_K, 16), f32)] * 5
            + [pltpu.VMEM((_K,), i32)] * 8
            + [pltpu.VMEM((32, HC), f32),
               pltpu.VMEM((32, 16), f32),
               pltpu.VMEM_SHARED((C_CAP, HC), f32),
               pltpu.VMEM_SHARED((C_CAP, 16), f32),
               pltpu.SemaphoreType.DMA]
        ),
        compiler_params=pltpu.CompilerParams(use_tc_tiling_on_sc=False, needs_layout_passes=False),
    )
    return f(pack, seg, xl, s16)


_BC = 512


def _combine_body(num_ref, den_ref, bias_ref, out_ref):
    num = num_ref[0] + num_ref[1]
    den = den_ref[0] + den_ref[1]
    B = num.shape[0]
    db = jnp.concatenate(
        [jnp.broadcast_to(den[:, h:h + 1], (B, ENC)) for h in range(HEADS)],
        axis=1)
    out_ref[...] = num / (db + 1e-16) + bias_ref[...]


def _combine(num, den, bias, nout):
    NV = num.shape[1]
    nblk = pl.cdiv(nout, _BC)
    assert nblk * _BC <= NV
    out = pl.pallas_call(
        _combine_body,
        grid=(nblk,),
        in_specs=[
            pl.BlockSpec((2, _BC, HC), lambda i: (0, i, 0)),
            pl.BlockSpec((2, _BC, 16), lambda i: (0, i, 0)),
            pl.BlockSpec((1, HC), lambda i: (0, 0)),
        ],
        out_specs=pl.BlockSpec((_BC, HC), lambda i: (i, 0)),
        out_shape=jax.ShapeDtypeStruct((nblk * _BC, HC), jnp.float32),
    )(num, den, bias[None, :])
    return out[:nout]


def _gat_layer(x, packp, segp, w, a_src, a_dst, bias, nseg_real, C,
               nchunk, nout):
    xl, s16 = _gat_dense(x, w, a_src, a_dst)
    num, den = _gat_edge_sc(xl, s16, packp, segp, nseg_real, C, nchunk)
    return _combine(num, den, bias, nout)


_BD = 512


def _dec_body(xt_ref, wfc_ref, bfc_ref,
              wi0_ref, wh0_ref, b0_ref,
              wi1_ref, wh1_ref, b1_ref,
              wop_ref, bop_ref, out_ref):
    B = xt_ref.shape[0]
    enc = jnp.dot(xt_ref[...], wfc_ref[...], preferred_element_type=jnp.float32) + bfc_ref[...]
    enc = jnp.maximum(enc, 0.1 * enc)
    gx0 = jnp.dot(enc, wi0_ref[...], preferred_element_type=jnp.float32) + b0_ref[...]
    h1 = jnp.zeros((B, DEC), jnp.float32)
    c1 = jnp.zeros((B, DEC), jnp.float32)
    h2 = jnp.zeros((B, DEC), jnp.float32)
    c2 = jnp.zeros((B, DEC), jnp.float32)
    wop = wop_ref[...]
    bop = bop_ref[...]
    for t in range(OUT_LEN):
        g = gx0 + jnp.dot(h1, wh0_ref[...], preferred_element_type=jnp.float32)
        i = jax.nn.sigmoid(g[:, 0:DEC])
        f = jax.nn.sigmoid(g[:, DEC:2 * DEC])
        gg = jnp.tanh(g[:, 2 * DEC:3 * DEC])
        o = jax.nn.sigmoid(g[:, 3 * DEC:])
        c1 = f * c1 + i * gg
        h1 = o * jnp.tanh(c1)
        g = (jnp.dot(h1, wi1_ref[...], preferred_element_type=jnp.float32)
             + jnp.dot(h2, wh1_ref[...], preferred_element_type=jnp.float32) + b1_ref[...])
        i = jax.nn.sigmoid(g[:, 0:DEC])
        f = jax.nn.sigmoid(g[:, DEC:2 * DEC])
        gg = jnp.tanh(g[:, 2 * DEC:3 * DEC])
        o = jax.nn.sigmoid(g[:, 3 * DEC:])
        c2 = f * c2 + i * gg
        h2 = o * jnp.tanh(c2)
        out_ref[:, pl.ds(2 * t, 2)] = (
            jnp.dot(h2, wop, preferred_element_type=jnp.float32) + bop)


def _decoder(g2t, p):
    grid = pl.cdiv(G, _BD)
    full = lambda s: pl.BlockSpec(s, lambda i: tuple(0 for _ in s))
    out = pl.pallas_call(
        _dec_body,
        grid=(grid,),
        in_specs=[
            pl.BlockSpec((_BD, HC), lambda i: (i, 0)),
            full((HC, ENC)), full((1, ENC)),
            full((ENC, 4 * DEC)), full((DEC, 4 * DEC)), full((1, 4 * DEC)),
            full((DEC, 4 * DEC)), full((DEC, 4 * DEC)), full((1, 4 * DEC)),
            full((DEC, 2)), full((1, 2)),
        ],
        out_specs=pl.BlockSpec((_BD, 2 * OUT_LEN), lambda i: (i, 0)),
        out_shape=jax.ShapeDtypeStruct((G, 2 * OUT_LEN), jnp.float32),
    )(g2t, p["W_fc"], p["b_fc"][None, :],
      p["lstm0_Wi"], p["lstm0_Wh"], p["lstm0_b"][None, :],
      p["lstm1_Wi"], p["lstm1_Wh"], p["lstm1_b"][None, :],
      p["W_op"], p["b_op"][None, :])
    return out.reshape(G, OUT_LEN, 2)


def kernel(x, params, edge_index, batch, num_graphs):
    p = params
    x2d = x.reshape(N, T * D_IN)
    henc = _encoder(x2d, p)

    ei = edge_index.astype(jnp.int32)
    loop = jnp.arange(N, dtype=jnp.int32)
    src = jnp.concatenate([ei[0], loop])
    dst = jnp.concatenate([ei[1], loop])
    npad = _E2P - src.shape[0]
    packp = jnp.pad(src | (dst << 16), (0, npad + 16))
    segp1 = jnp.pad(dst, (0, npad), constant_values=_PAD_SEG)

    g1 = _gat_layer(henc, packp, segp1, p["gat1_W"], p["gat1_asrc"],
                    p["gat1_adst"], p["gat1_b"], nseg_real=N, C=2048,
                    nchunk=25, nout=N)

    b32 = batch.astype(jnp.int32)
    is_tgt = jnp.concatenate([jnp.ones((1,), jnp.bool_), b32[1:] != b32[:-1]])
    slot = jnp.where(is_tgt, b32, _PAD_SEG)
    seg2 = jnp.pad(slot[dst], (0, npad), constant_values=_PAD_SEG)
    g2t = _gat_layer(g1, packp, seg2, p["gat2_W"], p["gat2_asrc"],
                     p["gat2_adst"], p["gat2_b"], nseg_real=G, C=2048,
                     nchunk=1, nout=G)
    return _decoder(g2t, p)


# speedup vs baseline: 6.0178x; 1.0430x over previous
import functools

import jax
import jax.numpy as jnp
from jax import lax
from jax.experimental import pallas as pl
from jax.experimental.pallas import tpu as pltpu
from jax.experimental.pallas import tpu_sc as plsc

N = 50000
T = 16
D_IN = 2
EMB = 32
ENC = 64
HEADS = 3
DEC = 128
OUT_LEN = 25
HC = HEADS * ENC
G = 2048


_BE = 512


def _enc_body(x_ref, wip_ref, bip_ref,
              wir_ref, wiz_ref, win_ref,
              whr_ref, whz_ref, whn_ref,
              brz_ref, bin_ref, bhn_ref,
              wdyn_ref, bdyn_ref, out_ref):
    xb = x_ref[...]
    B = xb.shape[0]
    wip = wip_ref[...]
    bip = bip_ref[...]
    h = jnp.zeros((B, ENC), dtype=jnp.float32)
    for t in range(T):
        x0 = xb[:, 2 * t:2 * t + 1]
        x1 = xb[:, 2 * t + 1:2 * t + 2]
        emb = x0 * wip[0:1, :] + x1 * wip[1:2, :] + bip
        emb = jnp.maximum(emb, 0.1 * emb)
        r = jax.nn.sigmoid(jnp.dot(emb, wir_ref[...], preferred_element_type=jnp.float32)
                           + jnp.dot(h, whr_ref[...], preferred_element_type=jnp.float32)
                           + brz_ref[0:1, :])
        z = jax.nn.sigmoid(jnp.dot(emb, wiz_ref[...], preferred_element_type=jnp.float32)
                           + jnp.dot(h, whz_ref[...], preferred_element_type=jnp.float32)
                           + brz_ref[1:2, :])
        hn = jnp.dot(h, whn_ref[...], preferred_element_type=jnp.float32) + bhn_ref[...]
        xn = jnp.dot(emb, win_ref[...], preferred_element_type=jnp.float32) + bin_ref[...]
        n = jnp.tanh(xn + r * hn)
        h = (1.0 - z) * n + z * h
    out = jnp.dot(h, wdyn_ref[...], preferred_element_type=jnp.float32) + bdyn_ref[...]
    out_ref[...] = jnp.maximum(out, 0.1 * out)


def _encoder(x2d, p):
    grid = pl.cdiv(N, _BE)
    wi = p["gru_Wi"]
    wh = p["gru_Wh"]
    bi = p["gru_bi"]
    bh = p["gru_bh"]
    brz = jnp.stack([bi[0:ENC] + bh[0:ENC], bi[ENC:2 * ENC] + bh[ENC:2 * ENC]])
    full = lambda s: pl.BlockSpec(s, lambda i: tuple(0 for _ in s))
    return pl.pallas_call(
        _enc_body,
        grid=(grid,),
        in_specs=[
            pl.BlockSpec((_BE, T * D_IN), lambda i: (i, 0)),
            full((D_IN, EMB)), full((1, EMB)),
            full((EMB, ENC)), full((EMB, ENC)), full((EMB, ENC)),
            full((ENC, ENC)), full((ENC, ENC)), full((ENC, ENC)),
            full((2, ENC)), full((1, ENC)), full((1, ENC)),
            full((ENC, ENC)), full((1, ENC)),
        ],
        out_specs=pl.BlockSpec((_BE, ENC), lambda i: (i, 0)),
        out_shape=jax.ShapeDtypeStruct((N, ENC), jnp.float32),
    )(x2d, p["W_ip"], p["b_ip"][None, :],
      wi[:, 0:ENC], wi[:, ENC:2 * ENC], wi[:, 2 * ENC:],
      wh[:, 0:ENC], wh[:, ENC:2 * ENC], wh[:, 2 * ENC:],
      brz, bi[None, 2 * ENC:], bh[None, 2 * ENC:],
      p["W_dyn"], p["b_dyn"][None, :])


_BG = 1024


def _gat_dense_body(x_ref, w_ref, a_ref, xl_ref, sc_ref):
    xl = jnp.dot(x_ref[...], w_ref[...], preferred_element_type=jnp.float32)
    xl_ref[...] = xl
    sc_ref[...] = jnp.dot(xl, a_ref[...], preferred_element_type=jnp.float32)


def _gat_dense(x, w, a_src, a_dst):
    d_in = x.shape[1]
    amat = jnp.zeros((HC, 16), jnp.float32)
    for h in range(HEADS):
        amat = amat.at[h * ENC:(h + 1) * ENC, h].set(a_src[h])
        amat = amat.at[h * ENC:(h + 1) * ENC, h + 4].set(a_dst[h])
    grid = pl.cdiv(N, _BG)
    return pl.pallas_call(
        _gat_dense_body,
        grid=(grid,),
        in_specs=[
            pl.BlockSpec((_BG, d_in), lambda i: (i, 0)),
            pl.BlockSpec((d_in, HC), lambda i: (0, 0)),
            pl.BlockSpec((HC, 16), lambda i: (0, 0)),
        ],
        out_specs=[
            pl.BlockSpec((_BG, HC), lambda i: (i, 0)),
            pl.BlockSpec((_BG, 16), lambda i: (i, 0)),
        ],
        out_shape=[
            jax.ShapeDtypeStruct((N, HC), jnp.float32),
            jax.ShapeDtypeStruct((N, 16), jnp.float32),
        ],
    )(x, w, amat)


_EPT = 26624
_E2P = 32 * _EPT
_K = 64
_PAD_SEG = 1 << 20


def _gat_edge_sc(xl, s16, pack, seg, nseg_real, C, nchunk):
    NV = nchunk * C
    C_CAP = C + 256
    ZR = C_CAP // 32
    R = C // 32
    i32 = jnp.int32
    f32 = jnp.float32

    def splits(total, step):
        out = []
        while total > 0:
            out.append(min(step, total))
            total -= out[-1]
        return out

    zsplits = splits(ZR, 32)
    fsplits = splits(R, 128)
    mesh = plsc.VectorSubcoreMesh(core_axis_name="c", subcore_axis_name="s")

    def body(pack_hbm, seg_hbm, xl_hbm, s16_hbm,
             num_hbm, den_hbm,
             seg_all, comp_idx, pk_a, pk_b,
             rows_a, rows_b, sxs_a, sxs_b, sxd_a, sxd_b, wbuf_a, wbuf_b,
             idx_a, idx_b, gix_a, gix_b, si_a, si_b, di_a, di_b,
             zrows, zden,
             numacc, denacc, sem, sem2):
        ci = lax.axis_index("c")
        si = lax.axis_index("s")
        wid = ci * 16 + si
        iota = lax.iota(i32, 16)
        zf = jnp.zeros((16,), f32)

        pltpu.async_copy(seg_hbm.at[pl.ds(wid * _EPT, _EPT)],
                         seg_all.at[pl.ds(0, _EPT)], sem).wait()

        def zinit(r, _):
            rv = jnp.full((16,), r, i32)
            for kq in range(HC // 16):
                plsc.store_scatter(zrows, [rv, iota + 16 * kq], zf)
            plsc.store_scatter(zden, [rv, iota], zf)
            return 0
        lax.fori_loop(0, 32, zinit, 0)

        def winit(r, _):
            plsc.store_scatter(wbuf_a, [jnp.full((16,), r, i32), iota], zf)
            plsc.store_scatter(wbuf_b, [jnp.full((16,), r, i32), iota], zf)
            return 0
        lax.fori_loop(0, _K, winit, 0)

        def zero_acc():
            off = 0
            for n in zsplits:
                r0 = pl.multiple_of(wid * ZR + off, 8)
                pltpu.sync_copy(zrows.at[pl.ds(0, n)], numacc.at[pl.ds(r0, n)])
                pltpu.sync_copy(zden.at[pl.ds(0, n)], denacc.at[pl.ds(r0, n)])
                off += n

        zero_acc()
        plsc.subcore_barrier()

        def chunk(c, _):
            base = c * C
            hi = jnp.minimum(base + C, nseg_real)
            plsc.store_scatter(seg_all, [iota + _EPT],
                               jnp.full((16,), base + C, i32))

            def vec(v, cnt):
                o = pl.multiple_of(v * 16, 16)
                segv = seg_all[pl.ds(o, 16)]
                m = (segv >= base) & (segv < hi)
                mi = m.astype(i32)
                pos = cnt + plsc.cumsum(mi) - 1
                plsc.store_scatter(comp_idx, [pos], iota + o, mask=m)
                return cnt + jnp.sum(mi)

            cnt = lax.fori_loop(0, _EPT // 16, vec, jnp.int32(0))

            for k in range(2 * _K // 16):
                plsc.store_scatter(comp_idx, [cnt + iota + 16 * k],
                                   jnp.full((16,), _EPT, i32))

            def prep(o, idx_r, gix_r, pk_r):
                for k in range(_K // 16):
                    lv = comp_idx[pl.ds(o + 16 * k, 16)]
                    segv = plsc.load_gather(seg_all, [lv])
                    idx_r[pl.ds(16 * k, 16)] = segv - base
                    gix_r[pl.ds(16 * k, 16)] = lv + wid * _EPT
                return pltpu.async_copy(pack_hbm.at[gix_r], pk_r, sem)

            def row_gather(pk_r, si_r, di_r, rows_r, ss_r, sd_r):
                for k in range(_K // 16):
                    pv = pk_r[pl.ds(16 * k, 16)]
                    si_r[pl.ds(16 * k, 16)] = pv & 0xFFFF
                    di_r[pl.ds(16 * k, 16)] = lax.shift_right_logical(pv, 16)
                c1 = pltpu.async_copy(xl_hbm.at[si_r], rows_r, sem)
                c2 = pltpu.async_copy(s16_hbm.at[si_r], ss_r, sem)
                c3 = pltpu.async_copy(s16_hbm.at[di_r], sd_r, sem)
                return c1, c2, c3

            def compute(rows_r, ss_r, sd_r, wb_r):
                for g in range(_K // 16):
                    rv = iota + 16 * g
                    for h in range(HEADS):
                        ss = plsc.load_gather(ss_r, [rv, jnp.full((16,), h, i32)])
                        sd = plsc.load_gather(sd_r, [rv, jnp.full((16,), h + 4, i32)])
                        a = ss + sd
                        a = jnp.maximum(a, 0.2 * a)
                        plsc.store_scatter(wb_r, [rv, jnp.full((16,), h, i32)],
                                           jnp.exp(a))

                def scale(e, _):
                    ev = jnp.full((16,), e, i32)
                    for h in range(HEADS):
                        wb = plsc.load_gather(wb_r, [ev, jnp.full((16,), h, i32)])
                        for q in range(ENC // 16):
                            cv = iota + (ENC * h + 16 * q)
                            x = plsc.load_gather(rows_r, [ev, cv])
                            plsc.store_scatter(rows_r, [ev, cv], x * wb)
                    return 0
                lax.fori_loop(0, _K, scale, 0)

            def pair(p, _):
                o0 = pl.multiple_of(p * 2 * _K, _K)
                o1 = o0 + _K
                cp0 = prep(o0, idx_a, gix_a, pk_a)
                cp0.wait()
                g0 = row_gather(pk_a, si_a, di_a, rows_a, sxs_a, sxd_a)
                cp1 = prep(o1, idx_b, gix_b, pk_b)
                for h in g0:
                    h.wait()
                cp1.wait()
                g1 = row_gather(pk_b, si_b, di_b, rows_b, sxs_b, sxd_b)
                compute(rows_a, sxs_a, sxd_a, wbuf_a)
                a1 = pltpu.async_copy(rows_a, numacc.at[idx_a], sem2, add=True)
                a2 = pltpu.async_copy(wbuf_a, denacc.at[idx_a], sem2, add=True)
                for h in g1:
                    h.wait()
                compute(rows_b, sxs_b, sxd_b, wbuf_b)
                a1.wait()
                a2.wait()
                b1 = pltpu.async_copy(rows_b, numacc.at[idx_b], sem2, add=True)
                b2 = pltpu.async_copy(wbuf_b, denacc.at[idx_b], sem2, add=True)
                b1.wait()
                b2.wait()
                return 0

            lax.fori_loop(0, (cnt + 2 * _K - 1) // (2 * _K), pair, 0)
            plsc.subcore_barrier()

            off = 0
            for n in fsplits:
                r0 = pl.multiple_of(wid * R + off, 8)
                pltpu.sync_copy(numacc.at[pl.ds(r0, n)],
                                num_hbm.at[ci, pl.ds(base + r0, n)])
                pltpu.sync_copy(denacc.at[pl.ds(r0, n)],
                                den_hbm.at[ci, pl.ds(base + r0, n)])
                off += n
            plsc.subcore_barrier()
            zero_acc()
            plsc.subcore_barrier()
            return 0

        lax.fori_loop(0, nchunk, chunk, 0)

    f = pl.kernel(
        body,
        out_type=[
            jax.ShapeDtypeStruct((2, NV, HC), f32),
            jax.ShapeDtypeStruct((2, NV, 16), f32),
        ],
        mesh=mesh,
        scratch_types=(
            [pltpu.VMEM((_EPT + 16,), i32),
             pltpu.VMEM((_EPT + 2 * _K,), i32)]
            + [pltpu.VMEM((_K,), i32)] * 2
            + [pltpu.VMEM((_K, HC), f32)] * 2
            + [pltpu.VMEM((_K, 16), f32)] * 6
            + [pltpu.VMEM((_K,), i32)] * 8
            + [pltpu.VMEM((32, HC), f32),
               pltpu.VMEM((32, 16), f32),
               pltpu.VMEM_SHARED((C_CAP, HC), f32),
               pltpu.VMEM_SHARED((C_CAP, 16), f32),
               pltpu.SemaphoreType.DMA,
               pltpu.SemaphoreType.DMA]
        ),
        compiler_params=pltpu.CompilerParams(use_tc_tiling_on_sc=False, needs_layout_passes=False),
    )
    return f(pack, seg, xl, s16)


_BC = 512


def _combine_body(num_ref, den_ref, bias_ref, out_ref):
    num = num_ref[0] + num_ref[1]
    den = den_ref[0] + den_ref[1]
    B = num.shape[0]
    db = jnp.concatenate(
        [jnp.broadcast_to(den[:, h:h + 1], (B, ENC)) for h in range(HEADS)],
        axis=1)
    out_ref[...] = num / (db + 1e-16) + bias_ref[...]


def _combine(num, den, bias, nout):
    NV = num.shape[1]
    nblk = pl.cdiv(nout, _BC)
    assert nblk * _BC <= NV
    out = pl.pallas_call(
        _combine_body,
        grid=(nblk,),
        in_specs=[
            pl.BlockSpec((2, _BC, HC), lambda i: (0, i, 0)),
            pl.BlockSpec((2, _BC, 16), lambda i: (0, i, 0)),
            pl.BlockSpec((1, HC), lambda i: (0, 0)),
        ],
        out_specs=pl.BlockSpec((_BC, HC), lambda i: (i, 0)),
        out_shape=jax.ShapeDtypeStruct((nblk * _BC, HC), jnp.float32),
    )(num, den, bias[None, :])
    return out[:nout]


def _gat_layer(x, packp, segp, w, a_src, a_dst, bias, nseg_real, C,
               nchunk, nout):
    xl, s16 = _gat_dense(x, w, a_src, a_dst)
    num, den = _gat_edge_sc(xl, s16, packp, segp, nseg_real, C, nchunk)
    return _combine(num, den, bias, nout)


_BD = 512


def _dec_body(xt_ref, wfc_ref, bfc_ref,
              wi0_ref, wh0_ref, b0_ref,
              wi1_ref, wh1_ref, b1_ref,
              wop_ref, bop_ref, out_ref):
    B = xt_ref.shape[0]
    enc = jnp.dot(xt_ref[...], wfc_ref[...], preferred_element_type=jnp.float32) + bfc_ref[...]
    enc = jnp.maximum(enc, 0.1 * enc)
    gx0 = jnp.dot(enc, wi0_ref[...], preferred_element_type=jnp.float32) + b0_ref[...]
    h1 = jnp.zeros((B, DEC), jnp.float32)
    c1 = jnp.zeros((B, DEC), jnp.float32)
    h2 = jnp.zeros((B, DEC), jnp.float32)
    c2 = jnp.zeros((B, DEC), jnp.float32)
    wop = wop_ref[...]
    bop = bop_ref[...]
    for t in range(OUT_LEN):
        g = gx0 + jnp.dot(h1, wh0_ref[...], preferred_element_type=jnp.float32)
        i = jax.nn.sigmoid(g[:, 0:DEC])
        f = jax.nn.sigmoid(g[:, DEC:2 * DEC])
        gg = jnp.tanh(g[:, 2 * DEC:3 * DEC])
        o = jax.nn.sigmoid(g[:, 3 * DEC:])
        c1 = f * c1 + i * gg
        h1 = o * jnp.tanh(c1)
        g = (jnp.dot(h1, wi1_ref[...], preferred_element_type=jnp.float32)
             + jnp.dot(h2, wh1_ref[...], preferred_element_type=jnp.float32) + b1_ref[...])
        i = jax.nn.sigmoid(g[:, 0:DEC])
        f = jax.nn.sigmoid(g[:, DEC:2 * DEC])
        gg = jnp.tanh(g[:, 2 * DEC:3 * DEC])
        o = jax.nn.sigmoid(g[:, 3 * DEC:])
        c2 = f * c2 + i * gg
        h2 = o * jnp.tanh(c2)
        out_ref[:, pl.ds(2 * t, 2)] = (
            jnp.dot(h2, wop, preferred_element_type=jnp.float32) + bop)


def _decoder(g2t, p):
    grid = pl.cdiv(G, _BD)
    full = lambda s: pl.BlockSpec(s, lambda i: tuple(0 for _ in s))
    out = pl.pallas_call(
        _dec_body,
        grid=(grid,),
        in_specs=[
            pl.BlockSpec((_BD, HC), lambda i: (i, 0)),
            full((HC, ENC)), full((1, ENC)),
            full((ENC, 4 * DEC)), full((DEC, 4 * DEC)), full((1, 4 * DEC)),
            full((DEC, 4 * DEC)), full((DEC, 4 * DEC)), full((1, 4 * DEC)),
            full((DEC, 2)), full((1, 2)),
        ],
        out_specs=pl.BlockSpec((_BD, 2 * OUT_LEN), lambda i: (i, 0)),
        out_shape=jax.ShapeDtypeStruct((G, 2 * OUT_LEN), jnp.float32),
    )(g2t, p["W_fc"], p["b_fc"][None, :],
      p["lstm0_Wi"], p["lstm0_Wh"], p["lstm0_b"][None, :],
      p["lstm1_Wi"], p["lstm1_Wh"], p["lstm1_b"][None, :],
      p["W_op"], p["b_op"][None, :])
    return out.reshape(G, OUT_LEN, 2)


def kernel(x, params, edge_index, batch, num_graphs):
    p = params
    x2d = x.reshape(N, T * D_IN)
    henc = _encoder(x2d, p)

    ei = edge_index.astype(jnp.int32)
    loop = jnp.arange(N, dtype=jnp.int32)
    src = jnp.concatenate([ei[0], loop])
    dst = jnp.concatenate([ei[1], loop])
    npad = _E2P - src.shape[0]
    packp = jnp.pad(src | (dst << 16), (0, npad + 16))
    segp1 = jnp.pad(dst, (0, npad), constant_values=_PAD_SEG)

    g1 = _gat_layer(henc, packp, segp1, p["gat1_W"], p["gat1_asrc"],
                    p["gat1_adst"], p["gat1_b"], nseg_real=N, C=2048,
                    nchunk=25, nout=N)

    b32 = batch.astype(jnp.int32)
    is_tgt = jnp.concatenate([jnp.ones((1,), jnp.bool_), b32[1:] != b32[:-1]])
    slot = jnp.where(is_tgt, b32, _PAD_SEG)
    seg2 = jnp.pad(slot[dst], (0, npad), constant_values=_PAD_SEG)
    g2t = _gat_layer(g1, packp, seg2, p["gat2_W"], p["gat2_asrc"],
                     p["gat2_adst"], p["gat2_b"], nseg_real=G, C=2048,
                     nchunk=1, nout=G)
    return _decoder(g2t, p)
